# baseline (speedup 1.0000x reference)
import dataclasses
import functools

import jax
import jax.numpy as jnp
from jax import lax
from jax.experimental import pallas as pl
from jax.experimental.pallas import tpu as pltpu
from jax.experimental.pallas import tpu_sc as plsc

HIDDEN = 512
HALF = HIDDEN // 2
DEPTH = 4
LANES = 16


def _pack_bf16(y):
    lo = lax.bitcast_convert_type(y[:, :HALF], jnp.uint32)
    hi = lax.bitcast_convert_type(y[:, HALF:], jnp.uint32)

    def rne(u):
        return (u + jnp.uint32(0x7FFF) + ((u >> 16) & jnp.uint32(1))) >> 16

    packed = rne(lo) | (rne(hi) << 16)
    return lax.bitcast_convert_type(packed, jnp.int32)


def _unpack_bf16(p):
    u = lax.bitcast_convert_type(p, jnp.uint32)
    lo = lax.bitcast_convert_type(u << 16, jnp.float32).astype(jnp.bfloat16)
    hi = lax.bitcast_convert_type(u & jnp.uint32(0xFFFF0000), jnp.float32)
    return lo, hi.astype(jnp.bfloat16)


def _mm_in_body(x_ref, w_ref, inp_ref, msg_ref):
    acc = jnp.dot(x_ref[...], w_ref[...], preferred_element_type=jnp.float32)
    inp_ref[...] = _pack_bf16(acc)
    msg_ref[...] = _pack_bf16(jnp.maximum(acc, 0.0))


def _mm_in(f_bonds, w_i):
    n, k = f_bonds.shape
    blk = 1280
    out_sd = jax.ShapeDtypeStruct((n, HALF), jnp.int32)
    return pl.pallas_call(
        _mm_in_body,
        grid=(n // blk,),
        in_specs=[
            pl.BlockSpec((blk, k), lambda i: (i, 0)),
            pl.BlockSpec((k, HIDDEN), lambda i: (0, 0)),
        ],
        out_specs=[
            pl.BlockSpec((blk, HALF), lambda i: (i, 0)),
            pl.BlockSpec((blk, HALF), lambda i: (i, 0)),
        ],
        out_shape=[out_sd, out_sd],
    )(f_bonds, w_i)


def _mm_lin_body(p_ref, wlo_ref, whi_ref, o_ref):
    xlo, xhi = _unpack_bf16(p_ref[...])
    acc = jnp.dot(xlo, wlo_ref[...], preferred_element_type=jnp.float32)
    acc = acc + jnp.dot(xhi, whi_ref[...], preferred_element_type=jnp.float32)
    o_ref[...] = _pack_bf16(acc)


def _mm_lin(p, w_lo, w_hi, blk):
    n = p.shape[0]
    return pl.pallas_call(
        _mm_lin_body,
        grid=(n // blk,),
        in_specs=[
            pl.BlockSpec((blk, HALF), lambda i: (i, 0)),
            pl.BlockSpec((HALF, HIDDEN), lambda i: (0, 0)),
            pl.BlockSpec((HALF, HIDDEN), lambda i: (0, 0)),
        ],
        out_specs=pl.BlockSpec((blk, HALF), lambda i: (i, 0)),
        out_shape=jax.ShapeDtypeStruct((n, HALF), jnp.int32),
    )(p, w_lo, w_hi)


def _mm_relu_body(g_ref, inp_ref, o_ref):
    glo, ghi = _unpack_bf16(g_ref[...])
    ilo, ihi = _unpack_bf16(inp_ref[...])
    lo = jnp.maximum(ilo.astype(jnp.float32) + glo.astype(jnp.float32), 0.0)
    hi = jnp.maximum(ihi.astype(jnp.float32) + ghi.astype(jnp.float32), 0.0)
    u_lo = lax.bitcast_convert_type(lo, jnp.uint32)
    u_hi = lax.bitcast_convert_type(hi, jnp.uint32)

    def rne(u):
        return (u + jnp.uint32(0x7FFF) + ((u >> 16) & jnp.uint32(1))) >> 16

    o_ref[...] = lax.bitcast_convert_type(rne(u_lo) | (rne(u_hi) << 16), jnp.int32)


def _mm_relu(g, inp):
    n, hw = g.shape
    blk = 1280
    return pl.pallas_call(
        _mm_relu_body,
        grid=(n // blk,),
        in_specs=[
            pl.BlockSpec((blk, hw), lambda i: (i, 0)),
            pl.BlockSpec((blk, hw), lambda i: (i, 0)),
        ],
        out_specs=pl.BlockSpec((blk, hw), lambda i: (i, 0)),
        out_shape=jax.ShapeDtypeStruct((n, hw), jnp.int32),
    )(g, inp)


def _mm_out_body(fa_ref, am_ref, w1_ref, w2lo_ref, w2hi_ref, b_ref, o_ref):
    acc = jnp.dot(fa_ref[...], w1_ref[...], preferred_element_type=jnp.float32)
    alo, ahi = _unpack_bf16(am_ref[...])
    acc = acc + jnp.dot(alo, w2lo_ref[...], preferred_element_type=jnp.float32)
    acc = acc + jnp.dot(ahi, w2hi_ref[...], preferred_element_type=jnp.float32)
    o_ref[...] = jnp.maximum(acc + b_ref[...], 0.0)


def _mm_out(f_atoms, a_message, w1, w2lo, w2hi, b_o):
    n, fa = f_atoms.shape
    h = w1.shape[1]
    blk = 2000
    return pl.pallas_call(
        _mm_out_body,
        grid=(n // blk,),
        in_specs=[
            pl.BlockSpec((blk, fa), lambda i: (i, 0)),
            pl.BlockSpec((blk, HALF), lambda i: (i, 0)),
            pl.BlockSpec((fa, h), lambda i: (0, 0)),
            pl.BlockSpec((HALF, h), lambda i: (0, 0)),
            pl.BlockSpec((HALF, h), lambda i: (0, 0)),
            pl.BlockSpec((1, h), lambda i: (0, 0)),
        ],
        out_specs=pl.BlockSpec((blk, h), lambda i: (i, 0)),
        out_shape=jax.ShapeDtypeStruct((n, h), jnp.float32),
    )(f_atoms, a_message, w1, w2lo, w2hi, b_o.reshape(1, h))


_MESH = plsc.VectorSubcoreMesh(core_axis_name="c", subcore_axis_name="s")
_NB = 32
_NWORK = 32

_SC_PARAMS = pltpu.CompilerParams()
if "needs_layout_passes" in pltpu.CompilerParams.__dataclass_fields__:
    _SC_PARAMS = dataclasses.replace(_SC_PARAMS, needs_layout_passes=False)


def _bf(v):
    return plsc.bitcast(v, jnp.bfloat16)


def _sc_gathersum(message, a2b_flat, n_pad):
    hw = message.shape[1]
    per_w = n_pad // _NWORK
    ab = 2
    nr = ab * _NB
    ob = 8
    nblk = per_w // ab

    @functools.partial(
        pl.kernel,
        mesh=_MESH,
        out_type=jax.ShapeDtypeStruct((n_pad, hw), jnp.int32),
        compiler_params=_SC_PARAMS,
        scratch_types=[
            pltpu.VMEM((per_w * _NB,), jnp.int32),
            pltpu.VMEM((nr, hw), jnp.int32),
            pltpu.VMEM((nr, hw), jnp.int32),
            pltpu.VMEM((nr, hw), jnp.int32),
            pltpu.VMEM((nr, hw), jnp.int32),
            pltpu.VMEM((ob, hw), jnp.int32),
            pltpu.SemaphoreType.DMA,
            pltpu.SemaphoreType.DMA,
            pltpu.SemaphoreType.DMA,
            pltpu.SemaphoreType.DMA,
        ],
    )
    def k(msg_hbm, idx_hbm, out_hbm, idx_v, r0, r1, r2, r3, outb, s0, s1, s2, s3):
        wid = lax.axis_index("s") * 2 + lax.axis_index("c")
        abase = wid * per_w
        bufs = ((r0, s0), (r1, s1), (r2, s2), (r3, s3))
        pltpu.sync_copy(idx_hbm.at[pl.ds(abase * _NB, per_w * _NB)], idx_v)
        for b in range(3):
            pltpu.async_copy(msg_hbm.at[idx_v.at[pl.ds(b * nr, nr)]], *bufs[b])

        @pl.loop(0, nblk, step=4)
        def _quad(i):
            for par in range(4):
                rcur, scur = bufs[par]
                rnxt, snxt = bufs[(par + 3) % 4]
                bb = i + par

                @pl.when(bb + 3 < nblk)
                def _issue():
                    pltpu.async_copy(
                        msg_hbm.at[idx_v.at[pl.ds((bb + 3) * nr, nr)]], rnxt, snxt
                    )

                pltpu.make_async_copy(
                    msg_hbm.at[idx_v.at[pl.ds(bb * nr, nr)]], rcur, scur
                ).wait()

                for a in range(ab):
                    slot = par * ab + a

                    @pl.loop(0, hw, step=LANES)
                    def _col(c):
                        cs = pl.ds(c, LANES)
                        accs = [_bf(rcur[a * _NB + q, cs]) for q in range(4)]
                        for j in range(4, _NB, 4):
                            for q in range(4):
                                accs[q] = accs[q] + _bf(rcur[a * _NB + j + q, cs])
                        tot = (accs[0] + accs[1]) + (accs[2] + accs[3])
                        outb[slot, cs] = plsc.bitcast(tot, jnp.int32)

            row = pl.multiple_of(abase + i * ab, ob)
            pltpu.sync_copy(outb, out_hbm.at[pl.ds(row, ob)])

    return k(message, a2b_flat)


_BBLK = 40


def _sc_update(ah, mh, b2a1d, b2revb1d):
    n = mh.shape[0]
    hw = mh.shape[1]
    per_w = n // _NWORK
    nstep = per_w // _BBLK

    @functools.partial(
        pl.kernel,
        mesh=_MESH,
        out_type=jax.ShapeDtypeStruct((n, hw), jnp.int32),
        compiler_params=_SC_PARAMS,
        scratch_types=[
            pltpu.VMEM((per_w,), jnp.int32),
            pltpu.VMEM((per_w,), jnp.int32),
            pltpu.VMEM((_BBLK, hw), jnp.int32),
            pltpu.VMEM((_BBLK, hw), jnp.int32),
            pltpu.VMEM((_BBLK, hw), jnp.int32),
            pltpu.VMEM((_BBLK, hw), jnp.int32),
            pltpu.SemaphoreType.DMA,
            pltpu.SemaphoreType.DMA,
            pltpu.SemaphoreType.DMA,
            pltpu.SemaphoreType.DMA,
        ],
    )
    def k(
        ah_hbm, mh_hbm, ia_hbm, ir_hbm, out_hbm,
        ia_v, ir_v, a0, a1, m0, m1, sg0, sg1, so0, so1,
    ):
        wid = lax.axis_index("s") * 2 + lax.axis_index("c")
        base = wid * per_w
        pltpu.sync_copy(ia_hbm.at[pl.ds(base, per_w)], ia_v)
        pltpu.sync_copy(ir_hbm.at[pl.ds(base, per_w)], ir_v)

        def issue(bb, abuf, mbuf, sem):
            pltpu.async_copy(ah_hbm.at[ia_v.at[pl.ds(bb * _BBLK, _BBLK)]], abuf, sem)
            pltpu.async_copy(mh_hbm.at[ir_v.at[pl.ds(bb * _BBLK, _BBLK)]], mbuf, sem)

        def wait(bb, abuf, mbuf, sem):
            pltpu.make_async_copy(
                ah_hbm.at[ia_v.at[pl.ds(bb * _BBLK, _BBLK)]], abuf, sem
            ).wait()
            pltpu.make_async_copy(
                mh_hbm.at[ir_v.at[pl.ds(bb * _BBLK, _BBLK)]], mbuf, sem
            ).wait()

        issue(0, a0, m0, sg0)

        @pl.loop(0, nstep, step=2)
        def _pair(i):
            for par, (ac, mc, sgc, soc, an, mn, sgn, son) in enumerate(
                ((a0, m0, sg0, so0, a1, m1, sg1, so1),
                 (a1, m1, sg1, so1, a0, m0, sg0, so0))
            ):
                bb = i + par

                @pl.when(bb >= 1)
                def _drain_prev_out():
                    row = pl.multiple_of(base + (bb - 1) * _BBLK, 8)
                    pltpu.make_async_copy(
                        an, out_hbm.at[pl.ds(row, _BBLK)], son
                    ).wait()

                @pl.when(bb + 1 < nstep)
                def _issue_next():
                    issue(bb + 1, an, mn, sgn)

                wait(bb, ac, mc, sgc)

                @pl.loop(0, hw, step=LANES)
                def _col(c):
                    cs = pl.ds(c, LANES)
                    for r in range(_BBLK):
                        v = _bf(ac[r, cs]) - _bf(mc[r, cs])
                        ac[r, cs] = plsc.bitcast(v, jnp.int32)

                row = pl.multiple_of(base + bb * _BBLK, 8)
                pltpu.async_copy(ac, out_hbm.at[pl.ds(row, _BBLK)], soc)

        row = pl.multiple_of(base + (nstep - 1) * _BBLK, 8)
        pltpu.make_async_copy(a1, out_hbm.at[pl.ds(row, _BBLK)], so1).wait()

    return k(ah, mh, b2a1d, b2revb1d)


def kernel(f_atoms, f_bonds, a2b, b2a, b2revb, a_scope, W_i, W_h, W_o, b_o):
    n_atoms, atom_fdim = f_atoms.shape
    n_mols = a_scope.shape[0]
    h = W_i.shape[1]

    n_pad = 10240
    a2b32 = a2b.astype(jnp.int32)
    a2b_pad = jnp.zeros((n_pad, _NB), jnp.int32).at[:n_atoms].set(a2b32)
    a2b_flat = a2b_pad.reshape(n_pad * _NB)
    b2a1d = b2a.astype(jnp.int32)
    b2revb1d = b2revb.astype(jnp.int32)
    w_h_lo = W_h[:HALF].astype(jnp.bfloat16)
    w_h_hi = W_h[HALF:].astype(jnp.bfloat16)
    w_o2 = W_o[atom_fdim:]
    w_o2_lo = w_o2[:HALF].astype(jnp.bfloat16)
    w_o2_hi = w_o2[HALF:].astype(jnp.bfloat16)

    inp_p, message = _mm_in(f_bonds, W_i)
    for _ in range(DEPTH - 1):
        a_message = _sc_gathersum(message, a2b_flat, n_pad)
        mh = _mm_lin(message, w_h_lo, w_h_hi, 1280)
        ah = _mm_lin(a_message, w_h_lo, w_h_hi, 1280)
        g = _sc_update(ah, mh, b2a1d, b2revb1d)
        message = _mm_relu(g, inp_p)
    a_message = _sc_gathersum(message, a2b_flat, n_pad)[:n_atoms]

    atom_hiddens = _mm_out(
        f_atoms, a_message, W_o[:atom_fdim], w_o2_lo, w_o2_hi, b_o
    )
    return atom_hiddens.reshape(n_mols, n_atoms // n_mols, h)


# speedup vs baseline: 1.0441x; 1.0441x over previous
import dataclasses
import functools

import jax
import jax.numpy as jnp
from jax import lax
from jax.experimental import pallas as pl
from jax.experimental.pallas import tpu as pltpu
from jax.experimental.pallas import tpu_sc as plsc

HIDDEN = 512
HALF = HIDDEN // 2
DEPTH = 4
LANES = 16


def _pack_bf16(y):
    lo = lax.bitcast_convert_type(y[:, :HALF], jnp.uint32)
    hi = lax.bitcast_convert_type(y[:, HALF:], jnp.uint32)

    def rne(u):
        return (u + jnp.uint32(0x7FFF) + ((u >> 16) & jnp.uint32(1))) >> 16

    packed = rne(lo) | (rne(hi) << 16)
    return lax.bitcast_convert_type(packed, jnp.int32)


def _unpack_bf16(p):
    u = lax.bitcast_convert_type(p, jnp.uint32)
    lo = lax.bitcast_convert_type(u << 16, jnp.float32).astype(jnp.bfloat16)
    hi = lax.bitcast_convert_type(u & jnp.uint32(0xFFFF0000), jnp.float32)
    return lo, hi.astype(jnp.bfloat16)


def _mm_in_body(x_ref, w_ref, inp_ref, msg_ref):
    acc = jnp.dot(x_ref[...], w_ref[...], preferred_element_type=jnp.float32)
    inp_ref[...] = _pack_bf16(acc)
    msg_ref[...] = _pack_bf16(jnp.maximum(acc, 0.0))


def _mm_in(f_bonds, w_i):
    n, k = f_bonds.shape
    blk = 1280
    out_sd = jax.ShapeDtypeStruct((n, HALF), jnp.int32)
    return pl.pallas_call(
        _mm_in_body,
        grid=(n // blk,),
        in_specs=[
            pl.BlockSpec((blk, k), lambda i: (i, 0)),
            pl.BlockSpec((k, HIDDEN), lambda i: (0, 0)),
        ],
        out_specs=[
            pl.BlockSpec((blk, HALF), lambda i: (i, 0)),
            pl.BlockSpec((blk, HALF), lambda i: (i, 0)),
        ],
        out_shape=[out_sd, out_sd],
    )(f_bonds, w_i)


def _mm_lin_body(p_ref, wlo_ref, whi_ref, o_ref):
    xlo, xhi = _unpack_bf16(p_ref[...])
    acc = jnp.dot(xlo, wlo_ref[...], preferred_element_type=jnp.float32)
    acc = acc + jnp.dot(xhi, whi_ref[...], preferred_element_type=jnp.float32)
    o_ref[...] = _pack_bf16(acc)


def _mm_lin(p, w_lo, w_hi, blk):
    n = p.shape[0]
    return pl.pallas_call(
        _mm_lin_body,
        grid=(n // blk,),
        in_specs=[
            pl.BlockSpec((blk, HALF), lambda i: (i, 0)),
            pl.BlockSpec((HALF, HIDDEN), lambda i: (0, 0)),
            pl.BlockSpec((HALF, HIDDEN), lambda i: (0, 0)),
        ],
        out_specs=pl.BlockSpec((blk, HALF), lambda i: (i, 0)),
        out_shape=jax.ShapeDtypeStruct((n, HALF), jnp.int32),
    )(p, w_lo, w_hi)


def _mm_out_body(fa_ref, am_ref, w1_ref, w2lo_ref, w2hi_ref, b_ref, o_ref):
    acc = jnp.dot(fa_ref[...], w1_ref[...], preferred_element_type=jnp.float32)
    alo, ahi = _unpack_bf16(am_ref[...])
    acc = acc + jnp.dot(alo, w2lo_ref[...], preferred_element_type=jnp.float32)
    acc = acc + jnp.dot(ahi, w2hi_ref[...], preferred_element_type=jnp.float32)
    o_ref[...] = jnp.maximum(acc + b_ref[...], 0.0)


def _mm_out(f_atoms, a_message, w1, w2lo, w2hi, b_o):
    n, fa = f_atoms.shape
    h = w1.shape[1]
    blk = 2000
    return pl.pallas_call(
        _mm_out_body,
        grid=(n // blk,),
        in_specs=[
            pl.BlockSpec((blk, fa), lambda i: (i, 0)),
            pl.BlockSpec((blk, HALF), lambda i: (i, 0)),
            pl.BlockSpec((fa, h), lambda i: (0, 0)),
            pl.BlockSpec((HALF, h), lambda i: (0, 0)),
            pl.BlockSpec((HALF, h), lambda i: (0, 0)),
            pl.BlockSpec((1, h), lambda i: (0, 0)),
        ],
        out_specs=pl.BlockSpec((blk, h), lambda i: (i, 0)),
        out_shape=jax.ShapeDtypeStruct((n, h), jnp.float32),
    )(f_atoms, a_message, w1, w2lo, w2hi, b_o.reshape(1, h))


_MESH = plsc.VectorSubcoreMesh(core_axis_name="c", subcore_axis_name="s")
_NB = 32
_NWORK = 32

_SC_PARAMS = pltpu.CompilerParams()
if "needs_layout_passes" in pltpu.CompilerParams.__dataclass_fields__:
    _SC_PARAMS = dataclasses.replace(_SC_PARAMS, needs_layout_passes=False)


def _bf(v):
    return plsc.bitcast(v, jnp.bfloat16)


def _sc_gathersum(message, a2b_flat, n_pad):
    hw = message.shape[1]
    per_w = n_pad // _NWORK
    ab = 2
    nr = ab * _NB
    ob = 8
    nblk = per_w // ab

    @functools.partial(
        pl.kernel,
        mesh=_MESH,
        out_type=jax.ShapeDtypeStruct((n_pad, hw), jnp.int32),
        compiler_params=_SC_PARAMS,
        scratch_types=[
            pltpu.VMEM((per_w * _NB,), jnp.int32),
            pltpu.VMEM((nr, hw), jnp.int32),
            pltpu.VMEM((nr, hw), jnp.int32),
            pltpu.VMEM((nr, hw), jnp.int32),
            pltpu.VMEM((nr, hw), jnp.int32),
            pltpu.VMEM((ob, hw), jnp.int32),
            pltpu.SemaphoreType.DMA,
            pltpu.SemaphoreType.DMA,
            pltpu.SemaphoreType.DMA,
            pltpu.SemaphoreType.DMA,
        ],
    )
    def k(msg_hbm, idx_hbm, out_hbm, idx_v, r0, r1, r2, r3, outb, s0, s1, s2, s3):
        wid = lax.axis_index("s") * 2 + lax.axis_index("c")
        abase = wid * per_w
        bufs = ((r0, s0), (r1, s1), (r2, s2), (r3, s3))
        pltpu.sync_copy(idx_hbm.at[pl.ds(abase * _NB, per_w * _NB)], idx_v)
        for b in range(3):
            pltpu.async_copy(msg_hbm.at[idx_v.at[pl.ds(b * nr, nr)]], *bufs[b])

        @pl.loop(0, nblk, step=4)
        def _quad(i):
            for par in range(4):
                rcur, scur = bufs[par]
                rnxt, snxt = bufs[(par + 3) % 4]
                bb = i + par

                @pl.when(bb + 3 < nblk)
                def _issue():
                    pltpu.async_copy(
                        msg_hbm.at[idx_v.at[pl.ds((bb + 3) * nr, nr)]], rnxt, snxt
                    )

                pltpu.make_async_copy(
                    msg_hbm.at[idx_v.at[pl.ds(bb * nr, nr)]], rcur, scur
                ).wait()

                for a in range(ab):
                    slot = par * ab + a

                    @pl.loop(0, hw, step=LANES)
                    def _col(c):
                        cs = pl.ds(c, LANES)
                        accs = [_bf(rcur[a * _NB + q, cs]) for q in range(4)]
                        for j in range(4, _NB, 4):
                            for q in range(4):
                                accs[q] = accs[q] + _bf(rcur[a * _NB + j + q, cs])
                        tot = (accs[0] + accs[1]) + (accs[2] + accs[3])
                        outb[slot, cs] = plsc.bitcast(tot, jnp.int32)

            row = pl.multiple_of(abase + i * ab, ob)
            pltpu.sync_copy(outb, out_hbm.at[pl.ds(row, ob)])

    return k(message, a2b_flat)


_BBLK = 40


def _sc_update(ah, mh, inp, b2a1d, b2revb1d):
    n = mh.shape[0]
    hw = mh.shape[1]
    per_w = n // _NWORK
    nstep = per_w // _BBLK

    @functools.partial(
        pl.kernel,
        mesh=_MESH,
        out_type=jax.ShapeDtypeStruct((n, hw), jnp.int32),
        compiler_params=_SC_PARAMS,
        scratch_types=[
            pltpu.VMEM((per_w,), jnp.int32),
            pltpu.VMEM((per_w,), jnp.int32),
            pltpu.VMEM((_BBLK, hw), jnp.int32),
            pltpu.VMEM((_BBLK, hw), jnp.int32),
            pltpu.VMEM((_BBLK, hw), jnp.int32),
            pltpu.VMEM((_BBLK, hw), jnp.int32),
            pltpu.VMEM((_BBLK, hw), jnp.int32),
            pltpu.VMEM((_BBLK, hw), jnp.int32),
            pltpu.SemaphoreType.DMA,
            pltpu.SemaphoreType.DMA,
            pltpu.SemaphoreType.DMA,
            pltpu.SemaphoreType.DMA,
        ],
    )
    def k(
        ah_hbm, mh_hbm, inp_hbm, ia_hbm, ir_hbm, out_hbm,
        ia_v, ir_v, a0, a1, m0, m1, p0, p1, sg0, sg1, so0, so1,
    ):
        wid = lax.axis_index("s") * 2 + lax.axis_index("c")
        base = wid * per_w
        pltpu.sync_copy(ia_hbm.at[pl.ds(base, per_w)], ia_v)
        pltpu.sync_copy(ir_hbm.at[pl.ds(base, per_w)], ir_v)

        def issue(bb, abuf, mbuf, pbuf, sem):
            pltpu.async_copy(ah_hbm.at[ia_v.at[pl.ds(bb * _BBLK, _BBLK)]], abuf, sem)
            pltpu.async_copy(mh_hbm.at[ir_v.at[pl.ds(bb * _BBLK, _BBLK)]], mbuf, sem)
            row = pl.multiple_of(base + bb * _BBLK, 8)
            pltpu.async_copy(inp_hbm.at[pl.ds(row, _BBLK)], pbuf, sem)

        def wait(bb, abuf, mbuf, pbuf, sem):
            pltpu.make_async_copy(
                ah_hbm.at[ia_v.at[pl.ds(bb * _BBLK, _BBLK)]], abuf, sem
            ).wait()
            pltpu.make_async_copy(
                mh_hbm.at[ir_v.at[pl.ds(bb * _BBLK, _BBLK)]], mbuf, sem
            ).wait()
            row = pl.multiple_of(base + bb * _BBLK, 8)
            pltpu.make_async_copy(inp_hbm.at[pl.ds(row, _BBLK)], pbuf, sem).wait()

        issue(0, a0, m0, p0, sg0)

        @pl.loop(0, nstep, step=2)
        def _pair(i):
            for par, (ac, mc, pc, sgc, soc, an, mn, pn, sgn, son) in enumerate(
                ((a0, m0, p0, sg0, so0, a1, m1, p1, sg1, so1),
                 (a1, m1, p1, sg1, so1, a0, m0, p0, sg0, so0))
            ):
                bb = i + par

                @pl.when(bb >= 1)
                def _drain_prev_out():
                    row = pl.multiple_of(base + (bb - 1) * _BBLK, 8)
                    pltpu.make_async_copy(
                        an, out_hbm.at[pl.ds(row, _BBLK)], son
                    ).wait()

                @pl.when(bb + 1 < nstep)
                def _issue_next():
                    issue(bb + 1, an, mn, pn, sgn)

                wait(bb, ac, mc, pc, sgc)

                @pl.loop(0, hw, step=LANES)
                def _col(c):
                    cs = pl.ds(c, LANES)
                    zero = jnp.zeros((2 * LANES,), jnp.bfloat16)
                    for r in range(_BBLK):
                        v = _bf(pc[r, cs]) + _bf(ac[r, cs]) - _bf(mc[r, cs])
                        ac[r, cs] = plsc.bitcast(jnp.maximum(v, zero), jnp.int32)

                row = pl.multiple_of(base + bb * _BBLK, 8)
                pltpu.async_copy(ac, out_hbm.at[pl.ds(row, _BBLK)], soc)

        row = pl.multiple_of(base + (nstep - 1) * _BBLK, 8)
        pltpu.make_async_copy(a1, out_hbm.at[pl.ds(row, _BBLK)], so1).wait()

    return k(ah, mh, inp, b2a1d, b2revb1d)


def kernel(f_atoms, f_bonds, a2b, b2a, b2revb, a_scope, W_i, W_h, W_o, b_o):
    n_atoms, atom_fdim = f_atoms.shape
    n_mols = a_scope.shape[0]
    h = W_i.shape[1]

    n_pad = 10240
    a2b32 = a2b.astype(jnp.int32)
    a2b_pad = jnp.zeros((n_pad, _NB), jnp.int32).at[:n_atoms].set(a2b32)
    a2b_flat = a2b_pad.reshape(n_pad * _NB)
    b2a1d = b2a.astype(jnp.int32)
    b2revb1d = b2revb.astype(jnp.int32)
    w_h_lo = W_h[:HALF].astype(jnp.bfloat16)
    w_h_hi = W_h[HALF:].astype(jnp.bfloat16)
    w_o2 = W_o[atom_fdim:]
    w_o2_lo = w_o2[:HALF].astype(jnp.bfloat16)
    w_o2_hi = w_o2[HALF:].astype(jnp.bfloat16)

    inp_p, message = _mm_in(f_bonds, W_i)
    for _ in range(DEPTH - 1):
        a_message = _sc_gathersum(message, a2b_flat, n_pad)
        mh = _mm_lin(message, w_h_lo, w_h_hi, 1280)
        ah = _mm_lin(a_message, w_h_lo, w_h_hi, 1280)
        message = _sc_update(ah, mh, inp_p, b2a1d, b2revb1d)
    a_message = _sc_gathersum(message, a2b_flat, n_pad)[:n_atoms]

    atom_hiddens = _mm_out(
        f_atoms, a_message, W_o[:atom_fdim], w_o2_lo, w_o2_hi, b_o
    )
    return atom_hiddens.reshape(n_mols, n_atoms // n_mols, h)
